# baseline (speedup 1.0000x reference)
import functools

import jax
import jax.numpy as jnp
from jax import lax
from jax.experimental import pallas as pl
from jax.experimental.pallas import tpu as pltpu
from jax.experimental.pallas import tpu_sc as plsc

N_NODES = 10000
N_EDGES = 320000
D = 128
EPS = 1e-5

NC = 2
NS = 16
NW = NC * NS
EPT = N_EDGES // NW
GC = 112
GN = 90
EPTP = GN * GC
NPAD = 10240
RPT = NPAD // NS
ZROWS = 128
CW = 128

def _sc_agg_body(x_hbm, ei_hbm, out_hbm,
                 sbuf, dbuf, didx0, rows0, rows1, acc, g0, g1):
    cid = lax.axis_index("c")
    sid = lax.axis_index("s")
    wid = sid * NC + cid
    zero16 = jnp.zeros((16,), jnp.float32)

    pltpu.sync_copy(ei_hbm.at[0, wid], sbuf)
    pltpu.sync_copy(ei_hbm.at[1, wid], dbuf)

    def zrow(i, _):
        for j in range(D // 16):
            rows0[i, pl.ds(j * 16, 16)] = zero16
        return 0
    lax.fori_loop(0, 80, zrow, 0)
    for q in range(RPT // 80):
        pltpu.sync_copy(rows0.at[pl.ds(0, 80)], acc.at[pl.ds(sid * RPT + q * 80, 80)])

    plsc.subcore_barrier()

    def fire(i, rows, sem):
        pltpu.async_copy(x_hbm.at[sbuf.at[pl.ds(i * GC, GC)]], rows, sem)

    def wait(rows, sem):
        pltpu.make_async_copy(x_hbm.at[sbuf.at[pl.ds(0, GC)]], rows, sem).wait()

    def scatter(i, rows):
        for k in range(GC // 16):
            didx0[pl.ds(k * 16, 16)] = dbuf[pl.ds(i * GC + k * 16, 16)]
        pltpu.sync_copy(rows, acc.at[didx0], add=True)

    fire(0, rows0, g0)
    fire(1, rows1, g1)

    def body(j, _):
        i0 = 2 * j
        wait(rows0, g0)
        scatter(i0, rows0)
        fire(i0 + 2, rows0, g0)
        wait(rows1, g1)
        scatter(i0 + 1, rows1)
        fire(i0 + 3, rows1, g1)
        return 0
    lax.fori_loop(0, GN // 2 - 1, body, 0)

    wait(rows0, g0)
    scatter(GN - 2, rows0)
    wait(rows1, g1)
    scatter(GN - 1, rows1)

    plsc.subcore_barrier()

    for q in range(RPT // 80):
        r0 = sid * RPT + q * 80
        pltpu.sync_copy(acc.at[pl.ds(r0, 80)], out_hbm.at[cid, pl.ds(r0, 80)])


def _sc_cnt_body(ei_hbm, cnt_hbm, dbuf, didx, ones, cacc, sem):
    cid = lax.axis_index("c")
    sid = lax.axis_index("s")
    wid = sid * NC + cid
    zero16 = jnp.zeros((16,), jnp.float32)
    one16 = jnp.ones((16,), jnp.float32)

    pltpu.sync_copy(ei_hbm.at[1, wid], dbuf)

    def orow(i, _):
        for j in range(CW // 16):
            ones[i, pl.ds(j * 16, 16)] = one16
        return 0
    lax.fori_loop(0, GC, orow, 0)

    def zrow(i, _):
        for j in range(CW // 16):
            ones[GC + i, pl.ds(j * 16, 16)] = zero16
        return 0
    lax.fori_loop(0, 80, zrow, 0)
    for q in range(RPT // 80):
        pltpu.sync_copy(ones.at[pl.ds(GC, 80)], cacc.at[pl.ds(sid * RPT + q * 80, 80)])

    plsc.subcore_barrier()

    def body(i, _):
        for k in range(GC // 16):
            didx[pl.ds(k * 16, 16)] = dbuf[pl.ds(i * GC + k * 16, 16)]
        pltpu.sync_copy(ones.at[pl.ds(0, GC)], cacc.at[didx], add=True)
        return 0
    lax.fori_loop(0, GN, body, 0)

    plsc.subcore_barrier()

    for q in range(RPT // 80):
        r0 = sid * RPT + q * 80
        pltpu.sync_copy(cacc.at[pl.ds(r0, 80)], cnt_hbm.at[cid, pl.ds(r0, 80)])


_SC_MESH = plsc.VectorSubcoreMesh(core_axis_name="c", subcore_axis_name="s")

_sc_agg = pl.kernel(
    _sc_agg_body,
    out_type=jax.ShapeDtypeStruct((NC, NPAD, D), jnp.float32),
    mesh=_SC_MESH,
    scratch_types=[
        pltpu.VMEM((EPTP,), jnp.int32),
        pltpu.VMEM((EPTP,), jnp.int32),
        pltpu.VMEM((GC,), jnp.int32),
        pltpu.VMEM((GC, D), jnp.float32),
        pltpu.VMEM((GC, D), jnp.float32),
        pltpu.VMEM_SHARED((NPAD, D), jnp.float32),
        pltpu.SemaphoreType.DMA,
        pltpu.SemaphoreType.DMA,
    ],
    name="sc_agg",
)

_sc_cnt = pl.kernel(
    _sc_cnt_body,
    out_type=jax.ShapeDtypeStruct((NC, NPAD, CW), jnp.float32),
    mesh=_SC_MESH,
    scratch_types=[
        pltpu.VMEM((EPTP,), jnp.int32),
        pltpu.VMEM((GC,), jnp.int32),
        pltpu.VMEM((GC + 80, CW), jnp.float32),
        pltpu.VMEM_SHARED((NPAD, CW), jnp.float32),
        pltpu.SemaphoreType.DMA,
    ],
    name="sc_cnt",
)


def _tc_prep_body(x_ref, m_ref, ei_ref, xo_ref, eo_ref):
    xo_ref[...] = x_ref[...]
    xo_ref[0:1, :] = m_ref[...]
    eo_ref[:, :, :EPT] = ei_ref[...]
    eo_ref[0:1, :, EPT:] = jnp.zeros((1, NW, EPTP - EPT), jnp.int32)
    eo_ref[1:2, :, EPT:] = jnp.full((1, NW, EPTP - EPT), NPAD - 1, jnp.int32)


_tc_prep = pl.pallas_call(
    _tc_prep_body,
    out_shape=[jax.ShapeDtypeStruct((N_NODES, D), jnp.float32),
               jax.ShapeDtypeStruct((2, NW, EPTP), jnp.int32)],
    name="tc_prep",
)


def _tc_stage_body(relu, p_ref, c_ref, x_ref, wl_ref, bl_ref, wr_ref,
                   g_ref, be_ref, o_ref):
    summed = (p_ref[0] + p_ref[1])[:N_NODES]
    cnt = (c_ref[0] + c_ref[1])[:N_NODES, 0:1]
    mean = summed * (1.0 / jnp.maximum(cnt, 1.0))
    h = (jnp.dot(mean, wl_ref[...], preferred_element_type=jnp.float32)
         + jnp.dot(x_ref[...], wr_ref[...], preferred_element_type=jnp.float32)
         + bl_ref[...])
    mu = jnp.mean(h, axis=0, keepdims=True)
    var = jnp.mean((h - mu) * (h - mu), axis=0, keepdims=True)
    h = (h - mu) * lax.rsqrt(var + EPS) * g_ref[...] + be_ref[...]
    if relu:
        h = jnp.maximum(h, 0.0)
    o_ref[...] = h + x_ref[...]


def _make_tc_stage(relu):
    return pl.pallas_call(
        functools.partial(_tc_stage_body, relu),
        out_shape=jax.ShapeDtypeStruct((N_NODES, D), jnp.float32),
        name="tc_stage_relu" if relu else "tc_stage",
    )


_tc_stage_relu = _make_tc_stage(True)
_tc_stage_last = _make_tc_stage(False)


def kernel(x, edge_index, mask_embed,
           Wl0, bl0, Wr0, g0, be0,
           Wl1, bl1, Wr1, g1, be1,
           Wl2, bl2, Wr2, g2, be2):
    ei3 = edge_index.reshape(2, NW, EPT)
    x0, eip = _tc_prep(x, mask_embed.reshape(1, D), ei3)

    convs = [(Wl0.T, bl0.reshape(1, D), Wr0.T, g0.reshape(1, D), be0.reshape(1, D)),
             (Wl1.T, bl1.reshape(1, D), Wr1.T, g1.reshape(1, D), be1.reshape(1, D)),
             (Wl2.T, bl2.reshape(1, D), Wr2.T, g2.reshape(1, D), be2.reshape(1, D))]

    cparts = _sc_cnt(eip)
    parts = _sc_agg(x0, eip)
    x1 = _tc_stage_relu(parts, cparts, x0, *convs[0])
    parts = _sc_agg(x1, eip)
    x2 = _tc_stage_relu(parts, cparts, x1, *convs[1])
    parts = _sc_agg(x2, eip)
    return _tc_stage_last(parts, cparts, x2, *convs[2])


# speedup vs baseline: 1.5240x; 1.5240x over previous
import functools

import jax
import jax.numpy as jnp
from jax import lax
from jax.experimental import pallas as pl
from jax.experimental.pallas import tpu as pltpu
from jax.experimental.pallas import tpu_sc as plsc

N_NODES = 10000
N_EDGES = 320000
D = 128
EPS = 1e-5

NC = 2
NS = 16
NW = NC * NS
EPT = N_EDGES // NW
CHUNK = 80
NCHUNK = EPT // CHUNK
GC = 80
GN = EPT // GC
NPAD = 10240
RPT = NPAD // NS
ZROWS = 128
CW = 128


def _sc_agg_body(x_hbm, ei_hbm, out_hbm,
                 sbuf, dbuf, didx, rows0, rows1, acc, g0, g1):
    cid = lax.axis_index("c")
    sid = lax.axis_index("s")
    wid = sid * NC + cid
    zero16 = jnp.zeros((16,), jnp.float32)

    pltpu.sync_copy(ei_hbm.at[0, wid], sbuf)
    pltpu.sync_copy(ei_hbm.at[1, wid], dbuf)

    def zrow(i, _):
        for j in range(D // 16):
            rows0[i, pl.ds(j * 16, 16)] = zero16
        return 0
    lax.fori_loop(0, GC, zrow, 0)
    for q in range(RPT // GC):
        pltpu.sync_copy(rows0, acc.at[pl.ds(sid * RPT + q * GC, GC)])

    plsc.subcore_barrier()

    def fire(i, rows, sem):
        pltpu.async_copy(x_hbm.at[sbuf.at[pl.ds(i * GC, GC)]], rows, sem)

    def wait(rows, sem):
        pltpu.make_async_copy(x_hbm.at[sbuf.at[pl.ds(0, GC)]], rows, sem).wait()

    def scatter(i, rows):
        for k in range(GC // 16):
            didx[pl.ds(k * 16, 16)] = dbuf[pl.ds(i * GC + k * 16, 16)]
        pltpu.sync_copy(rows, acc.at[didx], add=True)

    fire(0, rows0, g0)
    fire(1, rows1, g1)

    def body(j, _):
        i0 = 2 * j
        wait(rows0, g0)
        scatter(i0, rows0)
        fire(i0 + 2, rows0, g0)
        wait(rows1, g1)
        scatter(i0 + 1, rows1)
        fire(i0 + 3, rows1, g1)
        return 0
    lax.fori_loop(0, (GN - 3) // 2, body, 0)

    wait(rows0, g0)
    scatter(GN - 3, rows0)
    fire(GN - 1, rows0, g0)
    wait(rows1, g1)
    scatter(GN - 2, rows1)
    wait(rows0, g0)
    scatter(GN - 1, rows0)

    plsc.subcore_barrier()

    for q in range(RPT // GC):
        r0 = sid * RPT + q * GC
        pltpu.sync_copy(acc.at[pl.ds(r0, GC)], out_hbm.at[cid, pl.ds(r0, GC)])


def _sc_cnt_body(ei_hbm, cnt_hbm, dbuf, didx, ones, zc, cacc, sem):
    cid = lax.axis_index("c")
    sid = lax.axis_index("s")
    wid = sid * NC + cid
    zero16 = jnp.zeros((16,), jnp.float32)
    one16 = jnp.ones((16,), jnp.float32)

    pltpu.sync_copy(ei_hbm.at[1, wid], dbuf)

    def orow(i, _):
        for j in range(CW // 16):
            ones[i, pl.ds(j * 16, 16)] = one16
        return 0
    lax.fori_loop(0, CHUNK, orow, 0)

    def zcrow(i, _):
        for j in range(CW // 16):
            zc[i, pl.ds(j * 16, 16)] = zero16
        return 0
    lax.fori_loop(0, ZROWS, zcrow, 0)
    for q in range(RPT // ZROWS):
        pltpu.sync_copy(zc, cacc.at[pl.ds(sid * RPT + q * ZROWS, ZROWS)])

    plsc.subcore_barrier()

    def body(i, _):
        for k in range(CHUNK // 16):
            didx[pl.ds(k * 16, 16)] = dbuf[pl.ds(i * CHUNK + k * 16, 16)]
        pltpu.sync_copy(ones, cacc.at[didx], add=True)
        return 0
    lax.fori_loop(0, NCHUNK, body, 0)

    plsc.subcore_barrier()

    for q in range(RPT // ZROWS):
        r0 = sid * RPT + q * ZROWS
        pltpu.sync_copy(cacc.at[pl.ds(r0, ZROWS)], cnt_hbm.at[cid, pl.ds(r0, ZROWS)])


_SC_MESH = plsc.VectorSubcoreMesh(core_axis_name="c", subcore_axis_name="s")

_sc_agg = pl.kernel(
    _sc_agg_body,
    out_type=jax.ShapeDtypeStruct((NC, NPAD, D), jnp.float32),
    mesh=_SC_MESH,
    scratch_types=[
        pltpu.VMEM((EPT,), jnp.int32),
        pltpu.VMEM((EPT,), jnp.int32),
        pltpu.VMEM((GC,), jnp.int32),
        pltpu.VMEM((GC, D), jnp.float32),
        pltpu.VMEM((GC, D), jnp.float32),
        pltpu.VMEM_SHARED((NPAD, D), jnp.float32),
        pltpu.SemaphoreType.DMA,
        pltpu.SemaphoreType.DMA,
    ],
    name="sc_agg",
)

_sc_cnt = pl.kernel(
    _sc_cnt_body,
    out_type=jax.ShapeDtypeStruct((NC, NPAD, CW), jnp.float32),
    mesh=_SC_MESH,
    scratch_types=[
        pltpu.VMEM((EPT,), jnp.int32),
        pltpu.VMEM((CHUNK,), jnp.int32),
        pltpu.VMEM((CHUNK, CW), jnp.float32),
        pltpu.VMEM((ZROWS, CW), jnp.float32),
        pltpu.VMEM_SHARED((NPAD, CW), jnp.float32),
        pltpu.SemaphoreType.DMA,
    ],
    name="sc_cnt",
)


def _tc_mask_body(x_ref, m_ref, o_ref):
    o_ref[...] = x_ref[...]
    o_ref[0:1, :] = m_ref[...]


_tc_mask = pl.pallas_call(
    _tc_mask_body,
    out_shape=jax.ShapeDtypeStruct((N_NODES, D), jnp.float32),
    name="tc_mask",
)


def _tc_stage_body(relu, p_ref, c_ref, x_ref, wl_ref, bl_ref, wr_ref,
                   g_ref, be_ref, o_ref):
    summed = (p_ref[0] + p_ref[1])[:N_NODES]
    cnt = (c_ref[0] + c_ref[1])[:N_NODES, 0:1]
    mean = summed * (1.0 / jnp.maximum(cnt, 1.0))
    h = (jnp.dot(mean, wl_ref[...], preferred_element_type=jnp.float32)
         + jnp.dot(x_ref[...], wr_ref[...], preferred_element_type=jnp.float32)
         + bl_ref[...])
    mu = jnp.mean(h, axis=0, keepdims=True)
    var = jnp.mean((h - mu) * (h - mu), axis=0, keepdims=True)
    h = (h - mu) * lax.rsqrt(var + EPS) * g_ref[...] + be_ref[...]
    if relu:
        h = jnp.maximum(h, 0.0)
    o_ref[...] = h + x_ref[...]


def _make_tc_stage(relu):
    return pl.pallas_call(
        functools.partial(_tc_stage_body, relu),
        out_shape=jax.ShapeDtypeStruct((N_NODES, D), jnp.float32),
        name="tc_stage_relu" if relu else "tc_stage",
    )


_tc_stage_relu = _make_tc_stage(True)
_tc_stage_last = _make_tc_stage(False)


def kernel(x, edge_index, mask_embed,
           Wl0, bl0, Wr0, g0, be0,
           Wl1, bl1, Wr1, g1, be1,
           Wl2, bl2, Wr2, g2, be2):
    ei3 = edge_index.reshape(2, NW, EPT)
    x0 = _tc_mask(x, mask_embed.reshape(1, D))

    convs = [(Wl0.T, bl0.reshape(1, D), Wr0.T, g0.reshape(1, D), be0.reshape(1, D)),
             (Wl1.T, bl1.reshape(1, D), Wr1.T, g1.reshape(1, D), be1.reshape(1, D)),
             (Wl2.T, bl2.reshape(1, D), Wr2.T, g2.reshape(1, D), be2.reshape(1, D))]

    cparts = _sc_cnt(ei3)
    parts = _sc_agg(x0, ei3)
    x1 = _tc_stage_relu(parts, cparts, x0, *convs[0])
    parts = _sc_agg(x1, ei3)
    x2 = _tc_stage_relu(parts, cparts, x1, *convs[1])
    parts = _sc_agg(x2, ei3)
    return _tc_stage_last(parts, cparts, x2, *convs[2])
